# baseline (speedup 1.0000x reference)
import jax
import jax.numpy as jnp
from jax import lax
from jax.experimental import pallas as pl
from jax.experimental.pallas import tpu as pltpu
from jax.experimental.pallas import tpu_sc as plsc

N = 100000
C = 64
KVOL = 27
P = 30000
E = KVOL * P

NC = 2
NS = 16
NW = NC * NS

CH = 120
NCHUNK = E // CH
CPO = P // CH
CPH = CPO // 2
GROUPS = 4
GW = C // GROUPS
ROWS_PER_TILE = N // NS
ZROWS = 125
P_PAD = 30720
HALF = P_PAD // 2
PHALF = P // 2
PB = 2048
NB = P_PAD // PB
LINES = KVOL * HALF

SUP = 5
SPH = CPH // SUP
NSUP = KVOL * 2 * SPH


def _pair_line(c):
    k = c // CPO
    rc = c % CPO
    h = rc // CPH
    line = k * HALF + (rc - h * CPH) * CH
    return line, h


def _super_geom(s):
    k = s // (2 * SPH)
    rs = s % (2 * SPH)
    h = rs // SPH
    q = rs - h * SPH
    pair = k * P + h * PHALF + q * (SUP * CH)
    line = k * HALF + q * (SUP * CH)
    return pair, line, h


def _gather_body(kmap_hbm, feat_hbm, out_hbm,
                 idx_a, idx_b, rows_a, rows_b, sem_a, sem_b, sem_w):
    cid = lax.axis_index("c")
    sid = lax.axis_index("s")
    wid = sid * NC + cid
    nsup = (NSUP - wid + NW - 1) // NW

    def sup_id(i):
        return wid + i * NW

    def fire(i, idx_v, rows_v, sem):
        pair, _, _ = _super_geom(sup_id(i))
        pltpu.sync_copy(kmap_hbm.at[0, pl.ds(pair, SUP * CH)], idx_v)
        for j in range(SUP):
            pltpu.async_copy(feat_hbm.at[idx_v.at[pl.ds(j * CH, CH)]],
                             rows_v.at[pl.ds(j * CH, CH)], sem)

    def drain(idx_v, rows_v, sem):
        for j in range(SUP):
            pltpu.make_async_copy(feat_hbm.at[idx_v.at[pl.ds(j * CH, CH)]],
                                  rows_v.at[pl.ds(j * CH, CH)], sem).wait()

    def write(i, rows_v):
        _, line, h = _super_geom(sup_id(i))
        pltpu.sync_copy(rows_v,
                        out_hbm.at[pl.ds(line, SUP * CH), pl.ds(h * C, C)])

    fire(0, idx_a, rows_a, sem_a)

    def body(i, _):
        i0 = 2 * i
        fire(i0 + 1, idx_b, rows_b, sem_b)
        drain(idx_a, rows_a, sem_a)
        write(i0, rows_a)
        fire(jnp.minimum(i0 + 2, nsup - 1), idx_a, rows_a, sem_a)
        drain(idx_b, rows_b, sem_b)
        write(i0 + 1, rows_b)
        return 0

    lax.fori_loop(0, nsup // 2, body, 0)

    drain(idx_a, rows_a, sem_a)

    @pl.when(nsup % 2 == 1)
    def _():
        write(nsup - 1, rows_a)


def _sc_gather(kmap, feat):
    mesh = plsc.VectorSubcoreMesh(core_axis_name="c", subcore_axis_name="s")
    f = pl.kernel(
        _gather_body,
        out_type=jax.ShapeDtypeStruct((LINES, 2 * C), jnp.bfloat16),
        mesh=mesh,
        compiler_params=pltpu.CompilerParams(use_tc_tiling_on_sc=False),
        scratch_types=[
            pltpu.VMEM((SUP * CH,), jnp.int32),
            pltpu.VMEM((SUP * CH,), jnp.int32),
            pltpu.VMEM((SUP * CH, C), jnp.bfloat16),
            pltpu.VMEM((SUP * CH, C), jnp.bfloat16),
            pltpu.SemaphoreType.DMA,
            pltpu.SemaphoreType.DMA,
            pltpu.SemaphoreType.DMA,
        ],
    )
    return f(kmap, feat)


def _matmul_body(x_ref, w_ref, o_ref):
    o_ref[...] = jnp.dot(x_ref[...], w_ref[0],
                         preferred_element_type=jnp.float32)


def _tc_matmul(gathered, weights2):
    return pl.pallas_call(
        _matmul_body,
        grid=(KVOL, NB),
        in_specs=[
            pl.BlockSpec((PB // 2, 2 * C), lambda k, j: (k * NB + j, 0)),
            pl.BlockSpec((1, 2 * C, 2 * C), lambda k, j: (k, 0, 0)),
        ],
        out_specs=pl.BlockSpec((PB // 2, 2 * C), lambda k, j: (k * NB + j, 0)),
        out_shape=jax.ShapeDtypeStruct((LINES, 2 * C), jnp.float32),
    )(gathered, weights2)


def _scatter_body(kmap_hbm, msgs_hbm, out_hbm,
                  oidx_v, rows_v, zero_v, accum, sems, scat_sems):
    cid = lax.axis_index("c")
    sid = lax.axis_index("s")
    wid = sid
    nct = (NCHUNK - wid + NS - 1) // NS

    def zbody(i, _):
        zero_v[i, :] = jnp.zeros((GW,), jnp.float32)
        return 0
    lax.fori_loop(0, ZROWS, zbody, 0)

    for gi in range(2):
        g = cid * 2 + gi
        def zcopy(j, _):
            base = sid * ROWS_PER_TILE + j * ZROWS
            pltpu.sync_copy(zero_v, accum.at[pl.ds(base, ZROWS)])
            return 0
        lax.fori_loop(0, ROWS_PER_TILE // ZROWS, zcopy, 0)
        plsc.subcore_barrier()

        def chunk_id(i):
            return wid + i * NS

        def refs(i):
            c = chunk_id(i)
            line, h = _pair_line(c)
            return (kmap_hbm.at[1, pl.ds(c * CH, CH)],
                    msgs_hbm.at[pl.ds(line, CH), pl.ds(h * C + g * GW, GW)])

        def fire_loads(s, i):
            src_i, src_r = refs(i)
            pltpu.async_copy(src_i, oidx_v[s], sems[s])
            pltpu.async_copy(src_r, rows_v[s], sems[s])

        def drain_loads(s, i):
            src_i, src_r = refs(i)
            pltpu.make_async_copy(src_i, oidx_v[s], sems[s]).wait()
            pltpu.make_async_copy(src_r, rows_v[s], sems[s]).wait()

        def fire_scat(s):
            pltpu.async_copy(rows_v[s], accum.at[oidx_v[s]], scat_sems[s],
                             add=True)

        def drain_scat(s):
            pltpu.make_async_copy(rows_v[s], accum.at[oidx_v[s]],
                                  scat_sems[s]).wait()

        NSLOT = 6
        m = nct // NSLOT

        def body(i, _):
            for s in range(NSLOT):
                @pl.when(i > 0)
                def _():
                    drain_scat(s)
                fire_loads(s, i * NSLOT + s)
            for s in range(NSLOT):
                drain_loads(s, i * NSLOT + s)
                fire_scat(s)
            return 0

        lax.fori_loop(0, m, body, 0)

        for s in range(NSLOT):
            @pl.when(m > 0)
            def _():
                drain_scat(s)

        def tail(i, _):
            src_i, src_r = refs(i)
            pltpu.sync_copy(src_i, oidx_v[0])
            pltpu.sync_copy(src_r, rows_v[0])
            pltpu.sync_copy(rows_v[0], accum.at[oidx_v[0]], add=True)
            return 0

        lax.fori_loop(m * NSLOT, nct, tail, 0)

        plsc.subcore_barrier()

        rbase = sid * ROWS_PER_TILE
        pltpu.sync_copy(
            accum.at[pl.ds(rbase, ROWS_PER_TILE)],
            out_hbm.at[pl.ds(rbase, ROWS_PER_TILE), pl.ds(g * GW, GW)],
        )


def _sc_scatter(kmap, msgs):
    mesh = plsc.VectorSubcoreMesh(core_axis_name="c", subcore_axis_name="s")
    f = pl.kernel(
        _scatter_body,
        out_type=jax.ShapeDtypeStruct((N, C), jnp.float32),
        mesh=mesh,
        compiler_params=pltpu.CompilerParams(use_tc_tiling_on_sc=False),
        scratch_types=[
            [pltpu.VMEM((CH,), jnp.int32) for _ in range(6)],
            [pltpu.VMEM((CH, GW), jnp.float32) for _ in range(6)],
            pltpu.VMEM((ZROWS, GW), jnp.float32),
            pltpu.VMEM_SHARED((N, GW), jnp.float32),
            [pltpu.SemaphoreType.DMA for _ in range(6)],
            [pltpu.SemaphoreType.DMA for _ in range(6)],
        ],
    )
    return f(kmap, msgs)


@jax.jit
def kernel(input_feat, input_coord, input_cmap, input_kmap, kernel):
    weights = kernel
    wb = weights.astype(jnp.bfloat16)
    w2 = jnp.zeros((KVOL, 2 * C, 2 * C), jnp.bfloat16)
    w2 = w2.at[:, :C, :C].set(wb).at[:, C:, C:].set(wb)
    gathered = _sc_gather(input_kmap, input_feat.astype(jnp.bfloat16))
    msgs = _tc_matmul(gathered, w2)
    return _sc_scatter(input_kmap, msgs)


# speedup vs baseline: 1.4263x; 1.4263x over previous
import jax
import jax.numpy as jnp
from jax import lax
from jax.experimental import pallas as pl
from jax.experimental.pallas import tpu as pltpu
from jax.experimental.pallas import tpu_sc as plsc

N = 100000
C = 64
KVOL = 27
P = 30000
E = KVOL * P

NC = 2
NS = 16
NW = NC * NS

CH = 120
NCHUNK = E // CH
CPO = P // CH
CPH = CPO // 2
GROUPS = 4
GW = C // GROUPS
ROWS_PER_TILE = N // NS
ZROWS = 125
P_PAD = 30720
HALF = P_PAD // 2
PHALF = P // 2
PB = 2048
NB = P_PAD // PB
LINES = KVOL * HALF

SUP = 5
SPH = CPH // SUP
NSUP = KVOL * 2 * SPH


def _pair_line(c):
    k = c // CPO
    rc = c % CPO
    h = rc // CPH
    line = k * HALF + (rc - h * CPH) * CH
    return line, h


def _super_geom(s):
    k = s // (2 * SPH)
    rs = s % (2 * SPH)
    h = rs // SPH
    q = rs - h * SPH
    pair = k * P + h * PHALF + q * (SUP * CH)
    line = k * HALF + q * (SUP * CH)
    return pair, line, h


def _gather_body(kmap_hbm, feat_hbm, out_hbm,
                 idx_a, idx_b, rows_a, rows_b, sem_a, sem_b, sem_w):
    cid = lax.axis_index("c")
    sid = lax.axis_index("s")
    wid = sid * NC + cid
    nsup = (NSUP - wid + NW - 1) // NW

    def sup_id(i):
        return wid + i * NW

    def fire(i, idx_v, rows_v, sem):
        pair, _, _ = _super_geom(sup_id(i))
        pltpu.sync_copy(kmap_hbm.at[0, pl.ds(pair, SUP * CH)], idx_v)
        for j in range(SUP):
            pltpu.async_copy(feat_hbm.at[idx_v.at[pl.ds(j * CH, CH)]],
                             rows_v.at[pl.ds(j * CH, CH)], sem)

    def drain(idx_v, rows_v, sem):
        for j in range(SUP):
            pltpu.make_async_copy(feat_hbm.at[idx_v.at[pl.ds(j * CH, CH)]],
                                  rows_v.at[pl.ds(j * CH, CH)], sem).wait()

    def write(i, rows_v):
        _, line, h = _super_geom(sup_id(i))
        pltpu.sync_copy(rows_v,
                        out_hbm.at[pl.ds(line, SUP * CH), pl.ds(h * C, C)])

    fire(0, idx_a, rows_a, sem_a)

    def body(i, _):
        i0 = 2 * i
        fire(i0 + 1, idx_b, rows_b, sem_b)
        drain(idx_a, rows_a, sem_a)
        write(i0, rows_a)
        fire(jnp.minimum(i0 + 2, nsup - 1), idx_a, rows_a, sem_a)
        drain(idx_b, rows_b, sem_b)
        write(i0 + 1, rows_b)
        return 0

    lax.fori_loop(0, nsup // 2, body, 0)

    drain(idx_a, rows_a, sem_a)

    @pl.when(nsup % 2 == 1)
    def _():
        write(nsup - 1, rows_a)


def _sc_gather(kmap, feat):
    mesh = plsc.VectorSubcoreMesh(core_axis_name="c", subcore_axis_name="s")
    f = pl.kernel(
        _gather_body,
        out_type=jax.ShapeDtypeStruct((LINES, 2 * C), jnp.float32),
        mesh=mesh,
        compiler_params=pltpu.CompilerParams(use_tc_tiling_on_sc=False),
        scratch_types=[
            pltpu.VMEM((SUP * CH,), jnp.int32),
            pltpu.VMEM((SUP * CH,), jnp.int32),
            pltpu.VMEM((SUP * CH, C), jnp.float32),
            pltpu.VMEM((SUP * CH, C), jnp.float32),
            pltpu.SemaphoreType.DMA,
            pltpu.SemaphoreType.DMA,
            pltpu.SemaphoreType.DMA,
        ],
    )
    return f(kmap, feat)


def _matmul_body(x_ref, w_ref, o_ref):
    o_ref[...] = jnp.dot(x_ref[...], w_ref[0],
                         preferred_element_type=jnp.float32)


def _tc_matmul(gathered, weights2):
    return pl.pallas_call(
        _matmul_body,
        grid=(KVOL, NB),
        in_specs=[
            pl.BlockSpec((PB // 2, 2 * C), lambda k, j: (k * NB + j, 0)),
            pl.BlockSpec((1, 2 * C, 2 * C), lambda k, j: (k, 0, 0)),
        ],
        out_specs=pl.BlockSpec((PB // 2, 2 * C), lambda k, j: (k * NB + j, 0)),
        out_shape=jax.ShapeDtypeStruct((LINES, 2 * C), jnp.float32),
    )(gathered, weights2)


def _scatter_body(kmap_hbm, msgs_hbm, out_hbm,
                  oidx_v, rows_v, zero_v, accum, sems, scat_sems):
    cid = lax.axis_index("c")
    sid = lax.axis_index("s")
    wid = sid
    nct = (NCHUNK - wid + NS - 1) // NS

    def zbody(i, _):
        zero_v[i, :] = jnp.zeros((GW,), jnp.float32)
        return 0
    lax.fori_loop(0, ZROWS, zbody, 0)

    for gi in range(2):
        g = cid * 2 + gi
        def zcopy(j, _):
            base = sid * ROWS_PER_TILE + j * ZROWS
            pltpu.sync_copy(zero_v, accum.at[pl.ds(base, ZROWS)])
            return 0
        lax.fori_loop(0, ROWS_PER_TILE // ZROWS, zcopy, 0)
        plsc.subcore_barrier()

        def chunk_id(i):
            return wid + i * NS

        def refs(i):
            c = chunk_id(i)
            line, h = _pair_line(c)
            return (kmap_hbm.at[1, pl.ds(c * CH, CH)],
                    msgs_hbm.at[pl.ds(line, CH), pl.ds(h * C + g * GW, GW)])

        def fire_loads(s, i):
            src_i, src_r = refs(i)
            pltpu.async_copy(src_i, oidx_v[s], sems[s])
            pltpu.async_copy(src_r, rows_v[s], sems[s])

        def drain_loads(s, i):
            src_i, src_r = refs(i)
            pltpu.make_async_copy(src_i, oidx_v[s], sems[s]).wait()
            pltpu.make_async_copy(src_r, rows_v[s], sems[s]).wait()

        def fire_scat(s):
            pltpu.async_copy(rows_v[s], accum.at[oidx_v[s]], scat_sems[s],
                             add=True)

        def drain_scat(s):
            pltpu.make_async_copy(rows_v[s], accum.at[oidx_v[s]],
                                  scat_sems[s]).wait()

        NSLOT = 6
        m = nct // NSLOT

        def body(i, _):
            for s in range(NSLOT):
                @pl.when(i > 0)
                def _():
                    drain_scat(s)
                fire_loads(s, i * NSLOT + s)
            for s in range(NSLOT):
                drain_loads(s, i * NSLOT + s)
                fire_scat(s)
            return 0

        lax.fori_loop(0, m, body, 0)

        for s in range(NSLOT):
            @pl.when(m > 0)
            def _():
                drain_scat(s)

        def tail(i, _):
            src_i, src_r = refs(i)
            pltpu.sync_copy(src_i, oidx_v[0])
            pltpu.sync_copy(src_r, rows_v[0])
            pltpu.sync_copy(rows_v[0], accum.at[oidx_v[0]], add=True)
            return 0

        lax.fori_loop(m * NSLOT, nct, tail, 0)

        plsc.subcore_barrier()

        rbase = sid * ROWS_PER_TILE
        pltpu.sync_copy(
            accum.at[pl.ds(rbase, ROWS_PER_TILE)],
            out_hbm.at[pl.ds(rbase, ROWS_PER_TILE), pl.ds(g * GW, GW)],
        )


def _sc_scatter(kmap, msgs):
    mesh = plsc.VectorSubcoreMesh(core_axis_name="c", subcore_axis_name="s")
    f = pl.kernel(
        _scatter_body,
        out_type=jax.ShapeDtypeStruct((N, C), jnp.float32),
        mesh=mesh,
        compiler_params=pltpu.CompilerParams(use_tc_tiling_on_sc=False),
        scratch_types=[
            [pltpu.VMEM((CH,), jnp.int32) for _ in range(6)],
            [pltpu.VMEM((CH, GW), jnp.float32) for _ in range(6)],
            pltpu.VMEM((ZROWS, GW), jnp.float32),
            pltpu.VMEM_SHARED((N, GW), jnp.float32),
            [pltpu.SemaphoreType.DMA for _ in range(6)],
            [pltpu.SemaphoreType.DMA for _ in range(6)],
        ],
    )
    return f(kmap, msgs)


@jax.jit
def kernel(input_feat, input_coord, input_cmap, input_kmap, kernel):
    weights = kernel
    w2 = jnp.zeros((KVOL, 2 * C, 2 * C), jnp.float32)
    w2 = w2.at[:, :C, :C].set(weights).at[:, C:, C:].set(weights)
    gathered = _sc_gather(input_kmap, input_feat)
    msgs = _tc_matmul(gathered, w2)
    return _sc_scatter(input_kmap, msgs)


# speedup vs baseline: 1.5620x; 1.0951x over previous
import jax
import jax.numpy as jnp
from jax import lax
from jax.experimental import pallas as pl
from jax.experimental.pallas import tpu as pltpu
from jax.experimental.pallas import tpu_sc as plsc

N = 100000
C = 64
KVOL = 27
P = 30000
E = KVOL * P

NC = 2
NS = 16
NW = NC * NS

CH = 120
NCHUNK = E // CH
CPO = P // CH
CPH = CPO // 2
GROUPS = 4
GW = C // GROUPS
ROWS_PER_TILE = N // NS
ZROWS = 125
P_PAD = 30720
HALF = P_PAD // 2
PHALF = P // 2
PB = 3072
NB = P_PAD // PB
LINES = KVOL * HALF

SUP = 5
SPH = CPH // SUP
NSUP = KVOL * 2 * SPH


def _pair_line(c):
    k = c // CPO
    rc = c % CPO
    h = rc // CPH
    line = k * HALF + (rc - h * CPH) * CH
    return line, h


def _super_geom(s):
    k = s // (2 * SPH)
    rs = s % (2 * SPH)
    h = rs // SPH
    q = rs - h * SPH
    pair = k * P + h * PHALF + q * (SUP * CH)
    line = k * HALF + q * (SUP * CH)
    return pair, line, h


def _gather_body(kmap_hbm, feat_hbm, out_hbm,
                 idx_a, idx_b, rows_a, rows_b, sem_a, sem_b, sem_w):
    cid = lax.axis_index("c")
    sid = lax.axis_index("s")
    wid = sid * NC + cid
    nsup = (NSUP - wid + NW - 1) // NW

    def sup_id(i):
        return wid + i * NW

    def fire(i, idx_v, rows_v, sem):
        pair, _, _ = _super_geom(sup_id(i))
        pltpu.sync_copy(kmap_hbm.at[0, pl.ds(pair, SUP * CH)], idx_v)
        for j in range(SUP):
            pltpu.async_copy(feat_hbm.at[idx_v.at[pl.ds(j * CH, CH)]],
                             rows_v.at[pl.ds(j * CH, CH)], sem)

    def drain(idx_v, rows_v, sem):
        for j in range(SUP):
            pltpu.make_async_copy(feat_hbm.at[idx_v.at[pl.ds(j * CH, CH)]],
                                  rows_v.at[pl.ds(j * CH, CH)], sem).wait()

    def write(i, rows_v):
        _, line, h = _super_geom(sup_id(i))
        pltpu.sync_copy(rows_v,
                        out_hbm.at[pl.ds(line, SUP * CH), pl.ds(h * C, C)])

    fire(0, idx_a, rows_a, sem_a)

    def body(i, _):
        i0 = 2 * i
        fire(i0 + 1, idx_b, rows_b, sem_b)
        drain(idx_a, rows_a, sem_a)
        write(i0, rows_a)
        fire(jnp.minimum(i0 + 2, nsup - 1), idx_a, rows_a, sem_a)
        drain(idx_b, rows_b, sem_b)
        write(i0 + 1, rows_b)
        return 0

    lax.fori_loop(0, nsup // 2, body, 0)

    drain(idx_a, rows_a, sem_a)

    @pl.when(nsup % 2 == 1)
    def _():
        write(nsup - 1, rows_a)


def _sc_gather(kmap, feat):
    mesh = plsc.VectorSubcoreMesh(core_axis_name="c", subcore_axis_name="s")
    f = pl.kernel(
        _gather_body,
        out_type=jax.ShapeDtypeStruct((LINES, 2 * C), jnp.float32),
        mesh=mesh,
        compiler_params=pltpu.CompilerParams(use_tc_tiling_on_sc=False),
        scratch_types=[
            pltpu.VMEM((SUP * CH,), jnp.int32),
            pltpu.VMEM((SUP * CH,), jnp.int32),
            pltpu.VMEM((SUP * CH, C), jnp.float32),
            pltpu.VMEM((SUP * CH, C), jnp.float32),
            pltpu.SemaphoreType.DMA,
            pltpu.SemaphoreType.DMA,
            pltpu.SemaphoreType.DMA,
        ],
    )
    return f(kmap, feat)


def _matmul_body(x_ref, w_ref, o_ref):
    o_ref[...] = jnp.dot(x_ref[...], w_ref[0],
                         preferred_element_type=jnp.float32)


def _tc_matmul(gathered, weights2):
    return pl.pallas_call(
        _matmul_body,
        grid=(KVOL, NB),
        in_specs=[
            pl.BlockSpec((PB // 2, 2 * C), lambda k, j: (k * NB + j, 0)),
            pl.BlockSpec((1, 2 * C, 2 * C), lambda k, j: (k, 0, 0)),
        ],
        out_specs=pl.BlockSpec((PB // 2, 2 * C), lambda k, j: (k * NB + j, 0)),
        out_shape=jax.ShapeDtypeStruct((LINES, 2 * C), jnp.float32),
    )(gathered, weights2)


def _scatter_body(kmap_hbm, msgs_hbm, out_hbm,
                  oidx_v, rows_v, zero_v, accum, sems, scat_sems):
    cid = lax.axis_index("c")
    sid = lax.axis_index("s")
    wid = sid
    nct = (NCHUNK - wid + NS - 1) // NS

    def zbody(i, _):
        zero_v[i, :] = jnp.zeros((GW,), jnp.float32)
        return 0
    lax.fori_loop(0, ZROWS, zbody, 0)

    for gi in range(2):
        g = cid * 2 + gi
        def zcopy(j, _):
            base = sid * ROWS_PER_TILE + j * ZROWS
            pltpu.sync_copy(zero_v, accum.at[pl.ds(base, ZROWS)])
            return 0
        lax.fori_loop(0, ROWS_PER_TILE // ZROWS, zcopy, 0)
        plsc.subcore_barrier()

        def chunk_id(i):
            return wid + i * NS

        def refs(i):
            c = chunk_id(i)
            line, h = _pair_line(c)
            return (kmap_hbm.at[1, pl.ds(c * CH, CH)],
                    msgs_hbm.at[pl.ds(line, CH), pl.ds(h * C + g * GW, GW)])

        def fire_loads(s, i):
            src_i, src_r = refs(i)
            pltpu.async_copy(src_i, oidx_v[s], sems[s])
            pltpu.async_copy(src_r, rows_v[s], sems[s])

        def drain_loads(s, i):
            src_i, src_r = refs(i)
            pltpu.make_async_copy(src_i, oidx_v[s], sems[s]).wait()
            pltpu.make_async_copy(src_r, rows_v[s], sems[s]).wait()

        def fire_scat(s):
            pltpu.async_copy(rows_v[s], accum.at[oidx_v[s]], scat_sems[s],
                             add=True)

        def drain_scat(s):
            pltpu.make_async_copy(rows_v[s], accum.at[oidx_v[s]],
                                  scat_sems[s]).wait()

        NSLOT = 8
        m = nct // NSLOT

        def body(i, _):
            for s in range(NSLOT):
                @pl.when(i > 0)
                def _():
                    drain_scat(s)
                fire_loads(s, i * NSLOT + s)
            for s in range(NSLOT):
                drain_loads(s, i * NSLOT + s)
                fire_scat(s)
            return 0

        lax.fori_loop(0, m, body, 0)

        for s in range(NSLOT):
            @pl.when(m > 0)
            def _():
                drain_scat(s)

        def tail(i, _):
            src_i, src_r = refs(i)
            pltpu.sync_copy(src_i, oidx_v[0])
            pltpu.sync_copy(src_r, rows_v[0])
            pltpu.sync_copy(rows_v[0], accum.at[oidx_v[0]], add=True)
            return 0

        lax.fori_loop(m * NSLOT, nct, tail, 0)

        plsc.subcore_barrier()

        rbase = sid * ROWS_PER_TILE
        pltpu.sync_copy(
            accum.at[pl.ds(rbase, ROWS_PER_TILE)],
            out_hbm.at[pl.ds(rbase, ROWS_PER_TILE), pl.ds(g * GW, GW)],
        )


def _sc_scatter(kmap, msgs):
    mesh = plsc.VectorSubcoreMesh(core_axis_name="c", subcore_axis_name="s")
    f = pl.kernel(
        _scatter_body,
        out_type=jax.ShapeDtypeStruct((N, C), jnp.float32),
        mesh=mesh,
        compiler_params=pltpu.CompilerParams(use_tc_tiling_on_sc=False),
        scratch_types=[
            [pltpu.VMEM((CH,), jnp.int32) for _ in range(8)],
            [pltpu.VMEM((CH, GW), jnp.float32) for _ in range(8)],
            pltpu.VMEM((ZROWS, GW), jnp.float32),
            pltpu.VMEM_SHARED((N, GW), jnp.float32),
            [pltpu.SemaphoreType.DMA for _ in range(8)],
            [pltpu.SemaphoreType.DMA for _ in range(8)],
        ],
    )
    return f(kmap, msgs)


@jax.jit
def kernel(input_feat, input_coord, input_cmap, input_kmap, kernel):
    weights = kernel
    w2 = jnp.zeros((KVOL, 2 * C, 2 * C), jnp.float32)
    w2 = w2.at[:, :C, :C].set(weights).at[:, C:, C:].set(weights)
    gathered = _sc_gather(input_kmap, input_feat)
    msgs = _tc_matmul(gathered, w2)
    return _sc_scatter(input_kmap, msgs)


# speedup vs baseline: 1.6201x; 1.0372x over previous
import jax
import jax.numpy as jnp
from jax import lax
from jax.experimental import pallas as pl
from jax.experimental.pallas import tpu as pltpu
from jax.experimental.pallas import tpu_sc as plsc

N = 100000
C = 64
KVOL = 27
P = 30000
E = KVOL * P

NC = 2
NS = 16
NW = NC * NS

CH = 120
NCHUNK = E // CH
CPO = P // CH
CPH = CPO // 2
GROUPS = 4
GW = C // GROUPS
ROWS_PER_TILE = N // NS
ZROWS = 125
P_PAD = 30720
HALF = P_PAD // 2
PHALF = P // 2
PB = 3072
NB = P_PAD // PB
LINES = KVOL * HALF

SUP = 5
SPH = CPH // SUP
NSUP = KVOL * 2 * SPH


def _pair_line(c):
    k = c // CPO
    rc = c % CPO
    h = rc // CPH
    line = k * HALF + (rc - h * CPH) * CH
    return line, h


def _super_geom(s):
    k = s // (2 * SPH)
    rs = s % (2 * SPH)
    h = rs // SPH
    q = rs - h * SPH
    pair = k * P + h * PHALF + q * (SUP * CH)
    line = k * HALF + q * (SUP * CH)
    return pair, line, h


def _make_gather_body(k0, nk):
    nsup_range = nk * 2 * SPH

    def gather_body(kmap_hbm, feat_hbm, out_hbm,
                    idx_a, idx_b, rows_a, rows_b, sem_a, sem_b, sem_w):
        cid = lax.axis_index("c")
        sid = lax.axis_index("s")
        wid = sid * NC + cid
        nsup = (nsup_range - wid + NW - 1) // NW

        def geom(i):
            s = wid + i * NW
            k = s // (2 * SPH)
            rs = s % (2 * SPH)
            h = rs // SPH
            q = rs - h * SPH
            pair = (k0 + k) * P + h * PHALF + q * (SUP * CH)
            line = k * HALF + q * (SUP * CH)
            return pair, line, h

        def fire(i, idx_v, rows_v, sem):
            pair, _, _ = geom(i)
            pltpu.sync_copy(kmap_hbm.at[0, pl.ds(pair, SUP * CH)], idx_v)
            for j in range(SUP):
                pltpu.async_copy(feat_hbm.at[idx_v.at[pl.ds(j * CH, CH)]],
                                 rows_v.at[pl.ds(j * CH, CH)], sem)

        def drain(idx_v, rows_v, sem):
            for j in range(SUP):
                pltpu.make_async_copy(feat_hbm.at[idx_v.at[pl.ds(j * CH, CH)]],
                                      rows_v.at[pl.ds(j * CH, CH)], sem).wait()

        def write(i, rows_v):
            _, line, h = geom(i)
            pltpu.sync_copy(rows_v,
                            out_hbm.at[pl.ds(line, SUP * CH), pl.ds(h * C, C)])

        fire(0, idx_a, rows_a, sem_a)

        def body(i, _):
            i0 = 2 * i
            fire(i0 + 1, idx_b, rows_b, sem_b)
            drain(idx_a, rows_a, sem_a)
            write(i0, rows_a)
            fire(jnp.minimum(i0 + 2, nsup - 1), idx_a, rows_a, sem_a)
            drain(idx_b, rows_b, sem_b)
            write(i0 + 1, rows_b)
            return 0

        lax.fori_loop(0, nsup // 2, body, 0)

        drain(idx_a, rows_a, sem_a)

        @pl.when(nsup % 2 == 1)
        def _():
            write(nsup - 1, rows_a)

    return gather_body


def _sc_gather(kmap, feat, k0, nk):
    mesh = plsc.VectorSubcoreMesh(core_axis_name="c", subcore_axis_name="s")
    f = pl.kernel(
        _make_gather_body(k0, nk),
        out_type=jax.ShapeDtypeStruct((nk * HALF, 2 * C), jnp.float32),
        mesh=mesh,
        compiler_params=pltpu.CompilerParams(use_tc_tiling_on_sc=False),
        scratch_types=[
            pltpu.VMEM((SUP * CH,), jnp.int32),
            pltpu.VMEM((SUP * CH,), jnp.int32),
            pltpu.VMEM((SUP * CH, C), jnp.float32),
            pltpu.VMEM((SUP * CH, C), jnp.float32),
            pltpu.SemaphoreType.DMA,
            pltpu.SemaphoreType.DMA,
            pltpu.SemaphoreType.DMA,
        ],
    )
    return f(kmap, feat)


def _matmul_body(x_ref, w_ref, o_ref):
    o_ref[...] = jnp.dot(x_ref[...], w_ref[0],
                         preferred_element_type=jnp.float32)


def _matmul_body2(x_ref, w_ref, m_ref, o_ref):
    del m_ref
    o_ref[...] = jnp.dot(x_ref[...], w_ref[0],
                         preferred_element_type=jnp.float32)


KA = 14
KB = KVOL - KA


def _tc_matmul1(gathered_a, weights2):
    return pl.pallas_call(
        _matmul_body,
        grid=(KA, NB),
        in_specs=[
            pl.BlockSpec((PB // 2, 2 * C), lambda k, j: (k * NB + j, 0)),
            pl.BlockSpec((1, 2 * C, 2 * C), lambda k, j: (k, 0, 0)),
        ],
        out_specs=pl.BlockSpec((PB // 2, 2 * C), lambda k, j: (k * NB + j, 0)),
        out_shape=jax.ShapeDtypeStruct((LINES, 2 * C), jnp.float32),
    )(gathered_a, weights2)


def _tc_matmul2(gathered_b, weights2, msgs_in):
    return pl.pallas_call(
        _matmul_body2,
        grid=(KB, NB),
        in_specs=[
            pl.BlockSpec((PB // 2, 2 * C), lambda k, j: (k * NB + j, 0)),
            pl.BlockSpec((1, 2 * C, 2 * C), lambda k, j: (k + KA, 0, 0)),
            pl.BlockSpec(memory_space=pl.ANY),
        ],
        out_specs=pl.BlockSpec((PB // 2, 2 * C),
                               lambda k, j: ((k + KA) * NB + j, 0)),
        out_shape=jax.ShapeDtypeStruct((LINES, 2 * C), jnp.float32),
        input_output_aliases={2: 0},
    )(gathered_b, weights2, msgs_in)


def _scatter_body(kmap_hbm, msgs_hbm, out_hbm,
                  oidx_v, rows_v, zero_v, accum, sems, scat_sems):
    cid = lax.axis_index("c")
    sid = lax.axis_index("s")
    wid = sid
    nct = (NCHUNK - wid + NS - 1) // NS

    def zbody(i, _):
        zero_v[i, :] = jnp.zeros((GW,), jnp.float32)
        return 0
    lax.fori_loop(0, ZROWS, zbody, 0)

    for gi in range(2):
        g = cid * 2 + gi
        def zcopy(j, _):
            base = sid * ROWS_PER_TILE + j * ZROWS
            pltpu.sync_copy(zero_v, accum.at[pl.ds(base, ZROWS)])
            return 0
        lax.fori_loop(0, ROWS_PER_TILE // ZROWS, zcopy, 0)
        plsc.subcore_barrier()

        def chunk_id(i):
            return wid + i * NS

        def refs(i):
            c = chunk_id(i)
            line, h = _pair_line(c)
            return (kmap_hbm.at[1, pl.ds(c * CH, CH)],
                    msgs_hbm.at[pl.ds(line, CH), pl.ds(h * C + g * GW, GW)])

        def fire_loads(s, i):
            src_i, src_r = refs(i)
            pltpu.async_copy(src_i, oidx_v[s], sems[s])
            pltpu.async_copy(src_r, rows_v[s], sems[s])

        def drain_loads(s, i):
            src_i, src_r = refs(i)
            pltpu.make_async_copy(src_i, oidx_v[s], sems[s]).wait()
            pltpu.make_async_copy(src_r, rows_v[s], sems[s]).wait()

        def fire_scat(s):
            pltpu.async_copy(rows_v[s], accum.at[oidx_v[s]], scat_sems[s],
                             add=True)

        def drain_scat(s):
            pltpu.make_async_copy(rows_v[s], accum.at[oidx_v[s]],
                                  scat_sems[s]).wait()

        NSLOT = 8
        m = nct // NSLOT

        def body(i, _):
            for s in range(NSLOT):
                @pl.when(i > 0)
                def _():
                    drain_scat(s)
                fire_loads(s, i * NSLOT + s)
            for s in range(NSLOT):
                drain_loads(s, i * NSLOT + s)
                fire_scat(s)
            return 0

        lax.fori_loop(0, m, body, 0)

        for s in range(NSLOT):
            @pl.when(m > 0)
            def _():
                drain_scat(s)

        def tail(i, _):
            src_i, src_r = refs(i)
            pltpu.sync_copy(src_i, oidx_v[0])
            pltpu.sync_copy(src_r, rows_v[0])
            pltpu.sync_copy(rows_v[0], accum.at[oidx_v[0]], add=True)
            return 0

        lax.fori_loop(m * NSLOT, nct, tail, 0)

        plsc.subcore_barrier()

        rbase = sid * ROWS_PER_TILE
        pltpu.sync_copy(
            accum.at[pl.ds(rbase, ROWS_PER_TILE)],
            out_hbm.at[pl.ds(rbase, ROWS_PER_TILE), pl.ds(g * GW, GW)],
        )


def _sc_scatter(kmap, msgs):
    mesh = plsc.VectorSubcoreMesh(core_axis_name="c", subcore_axis_name="s")
    f = pl.kernel(
        _scatter_body,
        out_type=jax.ShapeDtypeStruct((N, C), jnp.float32),
        mesh=mesh,
        compiler_params=pltpu.CompilerParams(use_tc_tiling_on_sc=False),
        scratch_types=[
            [pltpu.VMEM((CH,), jnp.int32) for _ in range(8)],
            [pltpu.VMEM((CH, GW), jnp.float32) for _ in range(8)],
            pltpu.VMEM((ZROWS, GW), jnp.float32),
            pltpu.VMEM_SHARED((N, GW), jnp.float32),
            [pltpu.SemaphoreType.DMA for _ in range(8)],
            [pltpu.SemaphoreType.DMA for _ in range(8)],
        ],
    )
    return f(kmap, msgs)


@jax.jit
def kernel(input_feat, input_coord, input_cmap, input_kmap, kernel):
    weights = kernel
    w2 = jnp.zeros((KVOL, 2 * C, 2 * C), jnp.float32)
    w2 = w2.at[:, :C, :C].set(weights).at[:, C:, C:].set(weights)
    gathered_a = _sc_gather(input_kmap, input_feat, 0, KA)
    gathered_b = _sc_gather(input_kmap, input_feat, KA, KB)
    msgs1 = _tc_matmul1(gathered_a, w2)
    msgs = _tc_matmul2(gathered_b, w2, msgs1)
    return _sc_scatter(input_kmap, msgs)


# speedup vs baseline: 1.6748x; 1.0338x over previous
import jax
import jax.numpy as jnp
from jax import lax
from jax.experimental import pallas as pl
from jax.experimental.pallas import tpu as pltpu
from jax.experimental.pallas import tpu_sc as plsc

N = 100000
C = 64
KVOL = 27
P = 30000
E = KVOL * P

NC = 2
NS = 16
NW = NC * NS

CH = 120
NCHUNK = E // CH
CPO = P // CH
CPH = CPO // 2
GROUPS = 4
GW = C // GROUPS
ROWS_PER_TILE = N // NS
ZROWS = 125
P_PAD = 30720
HALF = P_PAD // 2
PHALF = P // 2
PB = 3072
NB = P_PAD // PB
LINES = KVOL * HALF

SUP = 5
SPH = CPH // SUP
NSUP = KVOL * 2 * SPH


def _pair_line(c):
    k = c // CPO
    rc = c % CPO
    h = rc // CPH
    line = k * HALF + (rc - h * CPH) * CH
    return line, h


def _super_geom(s):
    k = s // (2 * SPH)
    rs = s % (2 * SPH)
    h = rs // SPH
    q = rs - h * SPH
    pair = k * P + h * PHALF + q * (SUP * CH)
    line = k * HALF + q * (SUP * CH)
    return pair, line, h


def _make_gather_body(k0, nk):
    nsup_range = nk * 2 * SPH

    def gather_body(kmap_hbm, feat_hbm, out_hbm,
                    idx_a, idx_b, rows_a, rows_b, sem_a, sem_b, sem_w):
        cid = lax.axis_index("c")
        sid = lax.axis_index("s")
        wid = sid * NC + cid
        nsup = (nsup_range - wid + NW - 1) // NW

        def geom(i):
            s = wid + i * NW
            k = s // (2 * SPH)
            rs = s % (2 * SPH)
            h = rs // SPH
            q = rs - h * SPH
            pair = (k0 + k) * P + h * PHALF + q * (SUP * CH)
            line = k * HALF + q * (SUP * CH)
            return pair, line, h

        def fire(i, idx_v, rows_v, sem):
            pair, _, _ = geom(i)
            pltpu.sync_copy(kmap_hbm.at[0, pl.ds(pair, SUP * CH)], idx_v)
            for j in range(SUP):
                pltpu.async_copy(feat_hbm.at[idx_v.at[pl.ds(j * CH, CH)]],
                                 rows_v.at[pl.ds(j * CH, CH)], sem)

        def drain(idx_v, rows_v, sem):
            for j in range(SUP):
                pltpu.make_async_copy(feat_hbm.at[idx_v.at[pl.ds(j * CH, CH)]],
                                      rows_v.at[pl.ds(j * CH, CH)], sem).wait()

        def write(i, rows_v):
            _, line, h = geom(i)
            pltpu.sync_copy(rows_v,
                            out_hbm.at[pl.ds(line, SUP * CH), pl.ds(h * C, C)])

        fire(0, idx_a, rows_a, sem_a)

        def body(i, _):
            i0 = 2 * i
            fire(i0 + 1, idx_b, rows_b, sem_b)
            drain(idx_a, rows_a, sem_a)
            write(i0, rows_a)
            fire(jnp.minimum(i0 + 2, nsup - 1), idx_a, rows_a, sem_a)
            drain(idx_b, rows_b, sem_b)
            write(i0 + 1, rows_b)
            return 0

        lax.fori_loop(0, nsup // 2, body, 0)

        drain(idx_a, rows_a, sem_a)

        @pl.when(nsup % 2 == 1)
        def _():
            write(nsup - 1, rows_a)

    return gather_body


def _sc_gather(kmap, feat, k0, nk):
    mesh = plsc.VectorSubcoreMesh(core_axis_name="c", subcore_axis_name="s")
    f = pl.kernel(
        _make_gather_body(k0, nk),
        out_type=jax.ShapeDtypeStruct((nk * HALF, 2 * C), jnp.float32),
        mesh=mesh,
        compiler_params=pltpu.CompilerParams(use_tc_tiling_on_sc=False),
        scratch_types=[
            pltpu.VMEM((SUP * CH,), jnp.int32),
            pltpu.VMEM((SUP * CH,), jnp.int32),
            pltpu.VMEM((SUP * CH, C), jnp.float32),
            pltpu.VMEM((SUP * CH, C), jnp.float32),
            pltpu.SemaphoreType.DMA,
            pltpu.SemaphoreType.DMA,
            pltpu.SemaphoreType.DMA,
        ],
    )
    return f(kmap, feat)


def _matmul_body(x_ref, w_ref, o_ref):
    o_ref[...] = jnp.dot(x_ref[...], w_ref[0],
                         preferred_element_type=jnp.float32)


def _matmul_body2(x_ref, w_ref, m_ref, o_ref):
    del m_ref
    o_ref[...] = jnp.dot(x_ref[...], w_ref[0],
                         preferred_element_type=jnp.float32)


KR = 9


def _tc_matmul1(gathered_a, weights2):
    return pl.pallas_call(
        _matmul_body,
        grid=(KR, NB),
        in_specs=[
            pl.BlockSpec((PB // 2, 2 * C), lambda k, j: (k * NB + j, 0)),
            pl.BlockSpec((1, 2 * C, 2 * C), lambda k, j: (k, 0, 0)),
        ],
        out_specs=pl.BlockSpec((PB // 2, 2 * C), lambda k, j: (k * NB + j, 0)),
        out_shape=jax.ShapeDtypeStruct((LINES, 2 * C), jnp.float32),
    )(gathered_a, weights2)


def _tc_matmul2(gathered_b, weights2, msgs_in, k0):
    return pl.pallas_call(
        _matmul_body2,
        grid=(KR, NB),
        in_specs=[
            pl.BlockSpec((PB // 2, 2 * C), lambda k, j: (k * NB + j, 0)),
            pl.BlockSpec((1, 2 * C, 2 * C), lambda k, j, k0=k0: (k + k0, 0, 0)),
            pl.BlockSpec(memory_space=pl.ANY),
        ],
        out_specs=pl.BlockSpec((PB // 2, 2 * C),
                               lambda k, j, k0=k0: ((k + k0) * NB + j, 0)),
        out_shape=jax.ShapeDtypeStruct((LINES, 2 * C), jnp.float32),
        input_output_aliases={2: 0},
    )(gathered_b, weights2, msgs_in)


def _scatter_body(kmap_hbm, msgs_hbm, out_hbm,
                  oidx_v, rows_v, zero_v, accum, sems, scat_sems):
    cid = lax.axis_index("c")
    sid = lax.axis_index("s")
    wid = sid
    nct = (NCHUNK - wid + NS - 1) // NS

    def zbody(i, _):
        zero_v[i, :] = jnp.zeros((GW,), jnp.float32)
        return 0
    lax.fori_loop(0, ZROWS, zbody, 0)

    for gi in range(2):
        g = cid * 2 + gi
        def zcopy(j, _):
            base = sid * ROWS_PER_TILE + j * ZROWS
            pltpu.sync_copy(zero_v, accum.at[pl.ds(base, ZROWS)])
            return 0
        lax.fori_loop(0, ROWS_PER_TILE // ZROWS, zcopy, 0)
        plsc.subcore_barrier()

        def chunk_id(i):
            return wid + i * NS

        def refs(i):
            c = chunk_id(i)
            line, h = _pair_line(c)
            return (kmap_hbm.at[1, pl.ds(c * CH, CH)],
                    msgs_hbm.at[pl.ds(line, CH), pl.ds(h * C + g * GW, GW)])

        def fire_loads(s, i):
            src_i, src_r = refs(i)
            pltpu.async_copy(src_i, oidx_v[s], sems[s])
            pltpu.async_copy(src_r, rows_v[s], sems[s])

        def drain_loads(s, i):
            src_i, src_r = refs(i)
            pltpu.make_async_copy(src_i, oidx_v[s], sems[s]).wait()
            pltpu.make_async_copy(src_r, rows_v[s], sems[s]).wait()

        def fire_scat(s):
            pltpu.async_copy(rows_v[s], accum.at[oidx_v[s]], scat_sems[s],
                             add=True)

        def drain_scat(s):
            pltpu.make_async_copy(rows_v[s], accum.at[oidx_v[s]],
                                  scat_sems[s]).wait()

        NSLOT = 12
        m = nct // NSLOT

        def body(i, _):
            for s in range(NSLOT):
                @pl.when(i > 0)
                def _():
                    drain_scat(s)
                fire_loads(s, i * NSLOT + s)
            for s in range(NSLOT):
                drain_loads(s, i * NSLOT + s)
                fire_scat(s)
            return 0

        lax.fori_loop(0, m, body, 0)

        for s in range(NSLOT):
            @pl.when(m > 0)
            def _():
                drain_scat(s)

        def tail(i, _):
            src_i, src_r = refs(i)
            pltpu.sync_copy(src_i, oidx_v[0])
            pltpu.sync_copy(src_r, rows_v[0])
            pltpu.sync_copy(rows_v[0], accum.at[oidx_v[0]], add=True)
            return 0

        lax.fori_loop(m * NSLOT, nct, tail, 0)

        plsc.subcore_barrier()

        rbase = sid * ROWS_PER_TILE
        pltpu.sync_copy(
            accum.at[pl.ds(rbase, ROWS_PER_TILE)],
            out_hbm.at[pl.ds(rbase, ROWS_PER_TILE), pl.ds(g * GW, GW)],
        )


def _sc_scatter(kmap, msgs):
    mesh = plsc.VectorSubcoreMesh(core_axis_name="c", subcore_axis_name="s")
    f = pl.kernel(
        _scatter_body,
        out_type=jax.ShapeDtypeStruct((N, C), jnp.float32),
        mesh=mesh,
        compiler_params=pltpu.CompilerParams(use_tc_tiling_on_sc=False),
        scratch_types=[
            [pltpu.VMEM((CH,), jnp.int32) for _ in range(12)],
            [pltpu.VMEM((CH, GW), jnp.float32) for _ in range(12)],
            pltpu.VMEM((ZROWS, GW), jnp.float32),
            pltpu.VMEM_SHARED((N, GW), jnp.float32),
            [pltpu.SemaphoreType.DMA for _ in range(12)],
            [pltpu.SemaphoreType.DMA for _ in range(12)],
        ],
    )
    return f(kmap, msgs)


@jax.jit
def kernel(input_feat, input_coord, input_cmap, input_kmap, kernel):
    weights = kernel
    w2 = jnp.zeros((KVOL, 2 * C, 2 * C), jnp.float32)
    w2 = w2.at[:, :C, :C].set(weights).at[:, C:, C:].set(weights)
    gathered_a = _sc_gather(input_kmap, input_feat, 0, KR)
    gathered_b = _sc_gather(input_kmap, input_feat, KR, KR)
    gathered_c = _sc_gather(input_kmap, input_feat, 2 * KR, KR)
    msgs1 = _tc_matmul1(gathered_a, w2)
    msgs2 = _tc_matmul2(gathered_b, w2, msgs1, KR)
    msgs = _tc_matmul2(gathered_c, w2, msgs2, 2 * KR)
    return _sc_scatter(input_kmap, msgs)
